# Initial kernel scaffold; baseline (speedup 1.0000x reference)
import jax
import jax.numpy as jnp
from jax.experimental import pallas as pl


def kernel(x, incidence_indices, incidence_values, y, batch_0, W_node, b_node, W_edge, b_edge):
    raise NotImplementedError("write your pallas kernel here")


# speedup vs baseline: 3.1620x; 3.1620x over previous
import functools

import jax
import jax.numpy as jnp
from jax import lax
from jax.experimental import pallas as pl
from jax.experimental.pallas import tpu as pltpu
from jax.experimental.pallas import tpu_sc as plsc

_NE = 10000
_NC = 2
_NS = 16
_L = 16


def _sc_segment_sum(x2, row, col, val):
    half = x2.shape[1]
    e_total = val.shape[0]
    ept = e_total // _NS
    C = 128
    nfull = ept // C
    rem = ept - nfull * C
    zr = 104

    mesh = plsc.VectorSubcoreMesh(core_axis_name="c", subcore_axis_name="s")

    @functools.partial(
        pl.kernel,
        mesh=mesh,
        out_type=jax.ShapeDtypeStruct((_NC, _NE, half), jnp.float32),
        scratch_types=[
            pltpu.VMEM((1, C), jnp.int32),
            pltpu.VMEM((1, C), jnp.int32),
            pltpu.VMEM((C,), jnp.float32),
            pltpu.VMEM((1, C, half), jnp.float32),
            pltpu.VMEM((zr, half), jnp.float32),
            pltpu.VMEM_SHARED((_NE, half), jnp.float32),
            pltpu.SemaphoreType.DMA,
        ],
    )
    def seg_sum(x2_ref, row_ref, col_ref, val_ref, out_ref,
                rowbuf, colbuf, valbuf, stage, zbuf, acc, gsem):
        cid = lax.axis_index("c")
        sid = lax.axis_index("s")
        base = sid * ept
        rows0 = pl.multiple_of(sid * 624 + 8 * jnp.minimum(sid, 2), 8)
        has_extra = sid < 2

        zv = jnp.zeros((_L,), jnp.float32)

        def zero_row(i, carry):
            for j in range(half // _L):
                zbuf[i, pl.ds(j * _L, _L)] = zv
            return carry

        lax.fori_loop(0, zr, zero_row, 0)
        for k in range(624 // zr):
            pltpu.sync_copy(zbuf, acc.at[pl.ds(rows0 + k * zr, zr)])

        @pl.when(has_extra)
        def _():
            pltpu.sync_copy(zbuf.at[pl.ds(0, 8)],
                            acc.at[pl.ds(rows0 + 624, 8)])

        plsc.subcore_barrier()

        def do_chunk(off, cnt):
            pltpu.sync_copy(row_ref.at[pl.ds(off, cnt)],
                            rowbuf.at[0, pl.ds(0, cnt)])
            pltpu.sync_copy(col_ref.at[pl.ds(off, cnt)],
                            colbuf.at[0, pl.ds(0, cnt)])
            pltpu.sync_copy(val_ref.at[pl.ds(off, cnt)],
                            valbuf.at[pl.ds(0, cnt)])
            for j in range(cnt // _L):
                v = colbuf[0, pl.ds(j * _L, _L)]
                colbuf[0, pl.ds(j * _L, _L)] = v * 2 + cid
            pltpu.async_copy(
                x2_ref.at[colbuf.at[0, pl.ds(0, cnt)]],
                stage.at[0, pl.ds(0, cnt)], gsem).wait()

            def scale_group(g, carry):
                vv16 = valbuf[pl.ds(g * _L, _L)]
                for l in range(_L):
                    i = g * _L + l
                    vs = vv16[l]
                    for j in range(half // _L):
                        stage[0, i, pl.ds(j * _L, _L)] = (
                            stage[0, i, pl.ds(j * _L, _L)] * vs)
                return carry

            lax.fori_loop(0, cnt // _L, scale_group, 0)
            pltpu.sync_copy(stage.at[0, pl.ds(0, cnt)],
                            acc.at[rowbuf.at[0, pl.ds(0, cnt)]], add=True)

        def chunk_body(k, carry):
            do_chunk(pl.multiple_of(base + k * C, 8), C)
            return carry

        lax.fori_loop(0, nfull, chunk_body, 0)
        if rem:
            do_chunk(pl.multiple_of(base + nfull * C, 8), rem)

        plsc.subcore_barrier()
        pltpu.sync_copy(acc.at[pl.ds(rows0, 624)],
                        out_ref.at[cid, pl.ds(rows0, 624)])

        @pl.when(has_extra)
        def _():
            r1 = pl.multiple_of(rows0 + 624, 8)
            pltpu.sync_copy(acc.at[pl.ds(r1, 8)],
                            out_ref.at[cid, pl.ds(r1, 8)])

    return seg_sum(x2, row, col, val)


def _tc_dense(x, xh2, w_node, b_node, w_edge2, b_edge):
    n = x.shape[0]
    d = x.shape[1]
    h = w_node.shape[1]
    half = xh2.shape[2]
    R = 1000
    grid = (n // R,)

    def body(x_ref, xh_ref, wn_ref, bn_ref, we_ref, be_ref, on_ref, oe_ref):
        hn = jnp.dot(x_ref[...], wn_ref[...],
                     preferred_element_type=jnp.float32,
                     precision=lax.Precision.HIGHEST)
        on_ref[...] = jnp.maximum(hn + bn_ref[...], 0.0)
        he = (jnp.dot(xh_ref[0], we_ref[0],
                      preferred_element_type=jnp.float32,
                      precision=lax.Precision.HIGHEST)
              + jnp.dot(xh_ref[1], we_ref[1],
                        preferred_element_type=jnp.float32,
                        precision=lax.Precision.HIGHEST))
        oe_ref[...] = jnp.maximum(he + be_ref[...], 0.0)

    return pl.pallas_call(
        body,
        grid=grid,
        in_specs=[
            pl.BlockSpec((R, d), lambda i: (i, 0)),
            pl.BlockSpec((2, R, half), lambda i: (0, i, 0)),
            pl.BlockSpec((d, h), lambda i: (0, 0)),
            pl.BlockSpec((1, h), lambda i: (0, 0)),
            pl.BlockSpec((2, half, h), lambda i: (0, 0, 0)),
            pl.BlockSpec((1, h), lambda i: (0, 0)),
        ],
        out_specs=[
            pl.BlockSpec((R, h), lambda i: (i, 0)),
            pl.BlockSpec((R, h), lambda i: (i, 0)),
        ],
        out_shape=[
            jax.ShapeDtypeStruct((n, h), jnp.float32),
            jax.ShapeDtypeStruct((_NE, h), jnp.float32),
        ],
    )(x, xh2, w_node, b_node, w_edge2, b_edge)


def kernel(x, incidence_indices, incidence_values, y, batch_0,
           W_node, b_node, W_edge, b_edge):
    n, d = x.shape
    half = d // 2
    row = incidence_indices[0].astype(jnp.int32)
    col = incidence_indices[1].astype(jnp.int32)
    x2 = x.reshape(n * 2, half)
    xh2 = _sc_segment_sum(x2, row, col, incidence_values)
    w_edge2 = W_edge.reshape(2, half, W_edge.shape[1])
    xn, xe = _tc_dense(x, xh2, W_node, b_node.reshape(1, -1),
                       w_edge2, b_edge.reshape(1, -1))
    return (y, batch_0, xn, xe)


# speedup vs baseline: 4.9481x; 1.5648x over previous
import functools

import jax
import jax.numpy as jnp
from jax import lax
from jax.experimental import pallas as pl
from jax.experimental.pallas import tpu as pltpu
from jax.experimental.pallas import tpu_sc as plsc

_NE = 10000
_NC = 2
_NS = 16
_L = 16
_C = 128
_NB = 2


def _sc_segment_sum(x2, row, col, val):
    half = x2.shape[1]
    e_total = val.shape[0]
    nck_total = e_total // _C
    nck = nck_total // _NS
    n_extra = nck_total - nck * _NS
    assert nck % _NB == 0
    zr = 104

    mesh = plsc.VectorSubcoreMesh(core_axis_name="c", subcore_axis_name="s")

    @functools.partial(
        pl.kernel,
        mesh=mesh,
        out_type=jax.ShapeDtypeStruct((_NC, _NE, half), jnp.float32),
        scratch_types=[
            pltpu.VMEM((_NB, _C), jnp.int32),
            pltpu.VMEM((_NB, _C), jnp.int32),
            pltpu.VMEM((_NB, _C), jnp.float32),
            pltpu.VMEM((_NB, _C, half), jnp.float32),
            pltpu.VMEM((zr, half), jnp.float32),
            pltpu.VMEM_SHARED((_NE, half), jnp.float32),
            [pltpu.SemaphoreType.DMA] * _NB,
            [pltpu.SemaphoreType.DMA] * _NB,
            [pltpu.SemaphoreType.DMA] * _NB,
        ],
    )
    def seg_sum(x2_ref, row_ref, col_ref, val_ref, out_ref,
                rowbuf, colbuf, valbuf, stage, zbuf, acc,
                gsem, asem, isem):
        cid = lax.axis_index("c")
        sid = lax.axis_index("s")
        ck0 = sid * nck
        rows0 = pl.multiple_of(sid * 624 + 8 * jnp.minimum(sid, 2), 8)
        has_extra = sid < n_extra

        zv = jnp.zeros((_L,), jnp.float32)

        def zero_row(i, carry):
            for j in range(half // _L):
                zbuf[i, pl.ds(j * _L, _L)] = zv
            return carry

        lax.fori_loop(0, zr, zero_row, 0)
        for k in range(624 // zr):
            pltpu.sync_copy(zbuf, acc.at[pl.ds(rows0 + k * zr, zr)])

        @pl.when(has_extra)
        def _():
            pltpu.sync_copy(zbuf.at[pl.ds(0, 8)],
                            acc.at[pl.ds(rows0 + 624, 8)])

        plsc.subcore_barrier()

        def fire_idx(ck, b):
            off = pl.multiple_of(ck * _C, 8)
            pltpu.async_copy(row_ref.at[pl.ds(off, _C)], rowbuf.at[b],
                             isem[b])
            pltpu.async_copy(col_ref.at[pl.ds(off, _C)], colbuf.at[b],
                             isem[b])
            pltpu.async_copy(val_ref.at[pl.ds(off, _C)], valbuf.at[b],
                             isem[b])

        def wait_idx(ck, b):
            off = pl.multiple_of(ck * _C, 8)
            pltpu.make_async_copy(row_ref.at[pl.ds(off, _C)], rowbuf.at[b],
                                  isem[b]).wait()
            pltpu.make_async_copy(col_ref.at[pl.ds(off, _C)], colbuf.at[b],
                                  isem[b]).wait()
            pltpu.make_async_copy(val_ref.at[pl.ds(off, _C)], valbuf.at[b],
                                  isem[b]).wait()

        def transform_col(b):
            for j in range(_C // _L):
                v = colbuf[b, pl.ds(j * _L, _L)]
                colbuf[b, pl.ds(j * _L, _L)] = v * 2 + cid

        def fire_gather(b):
            pltpu.async_copy(x2_ref.at[colbuf.at[b]], stage.at[b], gsem[b])

        def wait_gather(b):
            pltpu.make_async_copy(x2_ref.at[colbuf.at[b]], stage.at[b],
                                  gsem[b]).wait()

        def scale(b):
            def scale_group(g, carry):
                vv16 = valbuf[b, pl.ds(g * _L, _L)]
                for l in range(_L):
                    i = g * _L + l
                    vs = vv16[l]
                    for j in range(half // _L):
                        stage[b, i, pl.ds(j * _L, _L)] = (
                            stage[b, i, pl.ds(j * _L, _L)] * vs)
                return carry

            lax.fori_loop(0, _C // _L, scale_group, 0)

        def fire_scatter(b):
            pltpu.async_copy(stage.at[b], acc.at[rowbuf.at[b]], asem[b],
                             add=True)

        def wait_scatter(b):
            pltpu.make_async_copy(stage.at[b], acc.at[rowbuf.at[b]],
                                  asem[b]).wait()

        fire_idx(ck0, 0)
        wait_idx(ck0, 0)
        transform_col(0)
        fire_gather(0)

        def step_body(p, carry):
            for b in range(_NB):
                k = p * _NB + b
                nb = (b + 1) % _NB
                if b == _NB - 1:
                    wait_scatter(nb)
                else:
                    @pl.when(k >= 2)
                    def _():
                        wait_scatter(nb)
                if b == _NB - 1:
                    @pl.when(p < nck // _NB - 1)
                    def _():
                        fire_idx(ck0 + k + 1, nb)
                else:
                    fire_idx(ck0 + k + 1, nb)
                wait_gather(b)
                scale(b)
                fire_scatter(b)
                if b == _NB - 1:
                    @pl.when(p < nck // _NB - 1)
                    def _():
                        wait_idx(ck0 + k + 1, nb)
                        transform_col(nb)
                        fire_gather(nb)
                else:
                    wait_idx(ck0 + k + 1, nb)
                    transform_col(nb)
                    fire_gather(nb)
            return carry

        lax.fori_loop(0, nck // _NB, step_body, 0)
        for j in range(1, _NB):
            wait_scatter((nck - j) % _NB)

        @pl.when(has_extra)
        def _():
            ck = nck * _NS + sid
            fire_idx(ck, 0)
            wait_idx(ck, 0)
            transform_col(0)
            fire_gather(0)
            wait_gather(0)
            scale(0)
            fire_scatter(0)
            wait_scatter(0)

        plsc.subcore_barrier()
        pltpu.sync_copy(acc.at[pl.ds(rows0, 624)],
                        out_ref.at[cid, pl.ds(rows0, 624)])

        @pl.when(has_extra)
        def _():
            r1 = pl.multiple_of(rows0 + 624, 8)
            pltpu.sync_copy(acc.at[pl.ds(r1, 8)],
                            out_ref.at[cid, pl.ds(r1, 8)])

    return seg_sum(x2, row, col, val)


def _tc_dense(x, xh2, w_node, b_node, w_edge2, b_edge):
    n = x.shape[0]
    d = x.shape[1]
    h = w_node.shape[1]
    half = xh2.shape[2]
    R = 1000
    grid = (n // R,)

    def body(x_ref, xh_ref, wn_ref, bn_ref, we_ref, be_ref, on_ref, oe_ref):
        hn = jnp.dot(x_ref[...], wn_ref[...],
                     preferred_element_type=jnp.float32,
                     precision=lax.Precision.HIGHEST)
        on_ref[...] = jnp.maximum(hn + bn_ref[...], 0.0)
        he = (jnp.dot(xh_ref[0], we_ref[0],
                      preferred_element_type=jnp.float32,
                      precision=lax.Precision.HIGHEST)
              + jnp.dot(xh_ref[1], we_ref[1],
                        preferred_element_type=jnp.float32,
                        precision=lax.Precision.HIGHEST))
        oe_ref[...] = jnp.maximum(he + be_ref[...], 0.0)

    return pl.pallas_call(
        body,
        grid=grid,
        in_specs=[
            pl.BlockSpec((R, d), lambda i: (i, 0)),
            pl.BlockSpec((2, R, half), lambda i: (0, i, 0)),
            pl.BlockSpec((d, h), lambda i: (0, 0)),
            pl.BlockSpec((1, h), lambda i: (0, 0)),
            pl.BlockSpec((2, half, h), lambda i: (0, 0, 0)),
            pl.BlockSpec((1, h), lambda i: (0, 0)),
        ],
        out_specs=[
            pl.BlockSpec((R, h), lambda i: (i, 0)),
            pl.BlockSpec((R, h), lambda i: (i, 0)),
        ],
        out_shape=[
            jax.ShapeDtypeStruct((n, h), jnp.float32),
            jax.ShapeDtypeStruct((_NE, h), jnp.float32),
        ],
    )(x, xh2, w_node, b_node, w_edge2, b_edge)


def kernel(x, incidence_indices, incidence_values, y, batch_0,
           W_node, b_node, W_edge, b_edge):
    n, d = x.shape
    half = d // 2
    row = incidence_indices[0].astype(jnp.int32)
    col = incidence_indices[1].astype(jnp.int32)
    x2 = x.reshape(n * 2, half)
    xh2 = _sc_segment_sum(x2, row, col, incidence_values)
    w_edge2 = W_edge.reshape(2, half, W_edge.shape[1])
    xn, xe = _tc_dense(x, xh2, W_node, b_node.reshape(1, -1),
                       w_edge2, b_edge.reshape(1, -1))
    return (y, batch_0, xn, xe)


# speedup vs baseline: 5.8560x; 1.1835x over previous
import functools

import jax
import jax.numpy as jnp
from jax import lax
from jax.experimental import pallas as pl
from jax.experimental.pallas import tpu as pltpu
from jax.experimental.pallas import tpu_sc as plsc

_NE = 10000
_NC = 2
_NS = 16
_L = 16
_C = 128
_NB = 2


def _sc_segment_sum(x, row, col, val):
    half = x.shape[1] // 2
    e_total = val.shape[0]
    nck_total = e_total // _C
    nck = nck_total // _NS
    n_extra = nck_total - nck * _NS
    assert nck % _NB == 0
    zr = 104

    mesh = plsc.VectorSubcoreMesh(core_axis_name="c", subcore_axis_name="s")

    @functools.partial(
        pl.kernel,
        mesh=mesh,
        out_type=jax.ShapeDtypeStruct((_NC, _NE, half), jnp.float32),
        scratch_types=[
            pltpu.VMEM((_NB, _C), jnp.int32),
            pltpu.VMEM((_NB, _C), jnp.int32),
            pltpu.VMEM((_NB, _C), jnp.float32),
            pltpu.VMEM((_NB, _C, half), jnp.float32),
            pltpu.VMEM((zr, half), jnp.float32),
            pltpu.VMEM_SHARED((_NE, half), jnp.float32),
            [pltpu.SemaphoreType.DMA] * _NB,
            [pltpu.SemaphoreType.DMA] * _NB,
            [pltpu.SemaphoreType.DMA] * _NB,
        ],
    )
    def seg_sum(x_ref, row_ref, col_ref, val_ref, out_ref,
                rowbuf, colbuf, valbuf, stage, zbuf, acc,
                gsem, asem, isem):
        cid = lax.axis_index("c")
        sid = lax.axis_index("s")
        col0 = pl.multiple_of(cid * half, 128)
        ck0 = sid * nck
        rows0 = pl.multiple_of(sid * 624 + 8 * jnp.minimum(sid, 2), 8)
        has_extra = sid < n_extra

        zv = jnp.zeros((_L,), jnp.float32)

        def zero_row(i, carry):
            for j in range(half // _L):
                zbuf[i, pl.ds(j * _L, _L)] = zv
            return carry

        lax.fori_loop(0, zr, zero_row, 0)
        for k in range(624 // zr):
            pltpu.sync_copy(zbuf, acc.at[pl.ds(rows0 + k * zr, zr)])

        @pl.when(has_extra)
        def _():
            pltpu.sync_copy(zbuf.at[pl.ds(0, 8)],
                            acc.at[pl.ds(rows0 + 624, 8)])

        plsc.subcore_barrier()

        def fire_idx(ck, b):
            off = pl.multiple_of(ck * _C, 8)
            pltpu.async_copy(row_ref.at[pl.ds(off, _C)], rowbuf.at[b],
                             isem[b])
            pltpu.async_copy(col_ref.at[pl.ds(off, _C)], colbuf.at[b],
                             isem[b])
            pltpu.async_copy(val_ref.at[pl.ds(off, _C)], valbuf.at[b],
                             isem[b])

        def wait_idx(ck, b):
            off = pl.multiple_of(ck * _C, 8)
            pltpu.make_async_copy(row_ref.at[pl.ds(off, _C)], rowbuf.at[b],
                                  isem[b]).wait()
            pltpu.make_async_copy(col_ref.at[pl.ds(off, _C)], colbuf.at[b],
                                  isem[b]).wait()
            pltpu.make_async_copy(val_ref.at[pl.ds(off, _C)], valbuf.at[b],
                                  isem[b]).wait()

        def fire_gather(b):
            pltpu.async_copy(x_ref.at[colbuf.at[b], pl.ds(col0, half)],
                             stage.at[b], gsem[b])

        def wait_gather(b):
            pltpu.make_async_copy(x_ref.at[colbuf.at[b], pl.ds(col0, half)],
                                  stage.at[b], gsem[b]).wait()

        def scale(b):
            def scale_group(g, carry):
                vv16 = valbuf[b, pl.ds(g * _L, _L)]
                for l in range(_L):
                    i = g * _L + l
                    vs = vv16[l]
                    for j in range(half // _L):
                        stage[b, i, pl.ds(j * _L, _L)] = (
                            stage[b, i, pl.ds(j * _L, _L)] * vs)
                return carry

            lax.fori_loop(0, _C // _L, scale_group, 0)

        def fire_scatter(b):
            pltpu.async_copy(stage.at[b], acc.at[rowbuf.at[b]], asem[b],
                             add=True)

        def wait_scatter(b):
            pltpu.make_async_copy(stage.at[b], acc.at[rowbuf.at[b]],
                                  asem[b]).wait()

        fire_idx(ck0, 0)
        wait_idx(ck0, 0)
        fire_gather(0)

        def step_body(p, carry):
            for b in range(_NB):
                k = p * _NB + b
                nb = (b + 1) % _NB
                if b == _NB - 1:
                    wait_scatter(nb)
                else:
                    @pl.when(k >= 2)
                    def _():
                        wait_scatter(nb)
                if b == _NB - 1:
                    @pl.when(p < nck // _NB - 1)
                    def _():
                        fire_idx(ck0 + k + 1, nb)
                else:
                    fire_idx(ck0 + k + 1, nb)
                wait_gather(b)
                scale(b)
                fire_scatter(b)
                if b == _NB - 1:
                    @pl.when(p < nck // _NB - 1)
                    def _():
                        wait_idx(ck0 + k + 1, nb)
                        fire_gather(nb)
                else:
                    wait_idx(ck0 + k + 1, nb)
                    fire_gather(nb)
            return carry

        lax.fori_loop(0, nck // _NB, step_body, 0)
        for j in range(1, _NB):
            wait_scatter((nck - j) % _NB)

        @pl.when(has_extra)
        def _():
            ck = nck * _NS + sid
            fire_idx(ck, 0)
            wait_idx(ck, 0)
            fire_gather(0)
            wait_gather(0)
            scale(0)
            fire_scatter(0)
            wait_scatter(0)

        plsc.subcore_barrier()
        pltpu.sync_copy(acc.at[pl.ds(rows0, 624)],
                        out_ref.at[cid, pl.ds(rows0, 624)])

        @pl.when(has_extra)
        def _():
            r1 = pl.multiple_of(rows0 + 624, 8)
            pltpu.sync_copy(acc.at[pl.ds(r1, 8)],
                            out_ref.at[cid, pl.ds(r1, 8)])

    return seg_sum(x, row, col, val)


def _tc_dense(x, xh2, w_node, b_node, w_edge, b_edge):
    n = x.shape[0]
    d = x.shape[1]
    h = w_node.shape[1]
    half = xh2.shape[2]
    R = 1000
    grid = (n // R,)

    def body(x_ref, xh_ref, wn_ref, bn_ref, we_ref, be_ref, on_ref, oe_ref):
        hn = jnp.dot(x_ref[...], wn_ref[...],
                     preferred_element_type=jnp.float32)
        on_ref[...] = jnp.maximum(hn + bn_ref[...], 0.0)
        we = we_ref[...]
        he = (jnp.dot(xh_ref[0], we[:half],
                      preferred_element_type=jnp.float32)
              + jnp.dot(xh_ref[1], we[half:],
                        preferred_element_type=jnp.float32))
        oe_ref[...] = jnp.maximum(he + be_ref[...], 0.0)

    return pl.pallas_call(
        body,
        grid=grid,
        in_specs=[
            pl.BlockSpec((R, d), lambda i: (i, 0)),
            pl.BlockSpec((2, R, half), lambda i: (0, i, 0)),
            pl.BlockSpec((d, h), lambda i: (0, 0)),
            pl.BlockSpec((1, h), lambda i: (0, 0)),
            pl.BlockSpec((d, h), lambda i: (0, 0)),
            pl.BlockSpec((1, h), lambda i: (0, 0)),
        ],
        out_specs=[
            pl.BlockSpec((R, h), lambda i: (i, 0)),
            pl.BlockSpec((R, h), lambda i: (i, 0)),
        ],
        out_shape=[
            jax.ShapeDtypeStruct((n, h), jnp.float32),
            jax.ShapeDtypeStruct((_NE, h), jnp.float32),
        ],
    )(x, xh2, w_node, b_node, w_edge, b_edge)


def kernel(x, incidence_indices, incidence_values, y, batch_0,
           W_node, b_node, W_edge, b_edge):
    n, d = x.shape
    half = d // 2
    row = incidence_indices[0].astype(jnp.int32)
    col = incidence_indices[1].astype(jnp.int32)
    xh2 = _sc_segment_sum(x, row, col, incidence_values)
    xn, xe = _tc_dense(x, xh2, W_node, b_node.reshape(1, -1),
                       W_edge, b_edge.reshape(1, -1))
    return (y, batch_0, xn, xe)
